# baseline (speedup 1.0000x reference)
import functools

import jax
import jax.numpy as jnp
from jax import lax
from jax.experimental import pallas as pl
from jax.experimental.pallas import tpu as pltpu
from jax.experimental.pallas import tpu_sc as plsc

N = 10000
E = 320000
D_IN = 128
HID = 64

NC = 2
NS = 16
TILES = NC * NS
C = 80
NCH = 126
CAP = NCH * C
SC_SIZES = [10080] * 11 + [9840] * 4 + [9760]
ROWS = 624
NACC = N + 8
TAIL = NACC - NS * ROWS
OTAIL = N - NS * ROWS

_mesh = plsc.VectorSubcoreMesh(core_axis_name="c", subcore_axis_name="s")


def _make_sc_segsum(D):
    @functools.partial(
        pl.kernel,
        out_type=jax.ShapeDtypeStruct((NC, N, D), jnp.float32),
        mesh=_mesh,
        compiler_params=pltpu.CompilerParams(use_tc_tiling_on_sc=False),
        scratch_types=[
            pltpu.VMEM((NCH, C), jnp.int32),
            pltpu.VMEM((NCH, C), jnp.int32),
            pltpu.VMEM((C, D), jnp.float32),
            pltpu.VMEM_SHARED((NACC, D), jnp.float32),
            pltpu.SemaphoreType.DMA,
        ],
    )
    def _sc_segsum(x_hbm, src_hbm, dst_hbm, zero_hbm, out_hbm,
                   src_v, dst_v, rows_v, acc_sh, sem):
        c = lax.axis_index("c")
        s = lax.axis_index("s")
        tid = c * NS + s
        pltpu.sync_copy(zero_hbm.at[pl.ds(s * ROWS, ROWS)],
                        acc_sh.at[pl.ds(s * ROWS, ROWS)])

        @pl.when(s == 0)
        def _():
            pltpu.sync_copy(zero_hbm.at[pl.ds(NS * ROWS, TAIL)],
                            acc_sh.at[pl.ds(NS * ROWS, TAIL)])

        pltpu.sync_copy(src_hbm.at[tid], src_v)
        pltpu.sync_copy(dst_hbm.at[tid], dst_v)
        plsc.subcore_barrier()

        def body(j, carry):
            pltpu.async_copy(x_hbm.at[src_v.at[j]], rows_v, sem).wait()
            pltpu.sync_copy(rows_v, acc_sh.at[dst_v.at[j]], add=True)
            return carry

        lax.fori_loop(0, NCH, body, 0)
        plsc.subcore_barrier()
        pltpu.sync_copy(acc_sh.at[pl.ds(s * ROWS, ROWS)],
                        out_hbm.at[c, pl.ds(s * ROWS, ROWS)])

        @pl.when(s == 0)
        def _():
            pltpu.sync_copy(acc_sh.at[pl.ds(NS * ROWS, OTAIL)],
                            out_hbm.at[c, pl.ds(NS * ROWS, OTAIL)])

    return _sc_segsum


_sc_segsum_in = _make_sc_segsum(D_IN)
_sc_segsum_hid = _make_sc_segsum(HID)


def _tc_mlp(x, parts, w1, b1, w2, b2):
    def body(x_ref, p_ref, w1_ref, b1_ref, w2_ref, b2_ref, o_ref):
        r = x_ref[...] + (p_ref[0] + p_ref[1])
        r1 = jnp.maximum(
            jnp.dot(r, w1_ref[...], preferred_element_type=jnp.float32)
            + b1_ref[...], 0.0)
        o_ref[...] = jnp.dot(r1, w2_ref[...],
                             preferred_element_type=jnp.float32) + b2_ref[...]

    return pl.pallas_call(
        body,
        out_shape=jax.ShapeDtypeStruct((N, HID), jnp.float32),
    )(x, parts, w1, b1, w2, b2)


def _tc_bn(r2, mean, var, gamma, beta):
    def body(r_ref, m_ref, v_ref, g_ref, be_ref, o_ref):
        xn = (g_ref[...] * (r_ref[...] - m_ref[...])
              * lax.rsqrt(v_ref[...] + 1e-5) + be_ref[...])
        o_ref[...] = jnp.maximum(xn, 0.0)

    return pl.pallas_call(
        body,
        out_shape=jax.ShapeDtypeStruct((N, HID), jnp.float32),
    )(r2, mean, var, gamma, beta)


def _prep_edges(edge_index):
    src, dst = edge_index[0], edge_index[1]
    order = jnp.argsort(dst, stable=True)
    ssrc, sdst = src[order], dst[order]
    srcs, dsts = [], []
    off = 0
    for sz in SC_SIZES * NC:
        pad = CAP - sz
        s_seg = lax.dynamic_slice_in_dim(ssrc, off, sz)
        d_seg = lax.dynamic_slice_in_dim(sdst, off, sz)
        if pad:
            s_seg = jnp.concatenate([s_seg, jnp.zeros((pad,), jnp.int32)])
            d_seg = jnp.concatenate(
                [d_seg, jnp.full((pad,), N, jnp.int32)])
        srcs.append(s_seg)
        dsts.append(d_seg)
        off += sz
    srcr = jnp.stack(srcs).reshape(TILES, NCH, C)
    dstr = jnp.stack(dsts).reshape(TILES, NCH, C)
    return srcr, dstr


def kernel(h, edge_index, params):
    srcr, dstr = _prep_edges(edge_index)
    zero_in = jnp.zeros((NACC, D_IN), jnp.float32)
    zero_hid = jnp.zeros((NACC, HID), jnp.float32)
    b1 = [b.reshape(1, HID) for b in params["b1"]]
    b2 = [b.reshape(1, HID) for b in params["b2"]]
    gamma = [g.reshape(1, HID) for g in params["gamma"]]
    beta = [b.reshape(1, HID) for b in params["beta"]]

    x = h
    for l in range(4):
        if l == 0:
            parts = _sc_segsum_in(x, srcr, dstr, zero_in)
        else:
            parts = _sc_segsum_hid(x, srcr, dstr, zero_hid)
        r2 = _tc_mlp(x, parts, params["W1"][l], b1[l], params["W2"][l],
                     b2[l])
        mean = jnp.mean(r2, axis=0).reshape(1, HID)
        var = jnp.var(r2, axis=0).reshape(1, HID)
        x = _tc_bn(r2, mean, var, gamma[l], beta[l])
    return x
